# baseline (speedup 1.0000x reference)
import jax
import jax.numpy as jnp
from jax import lax
from jax.experimental import pallas as pl

_MAX_LEN = 512
_D = 64
_PK = 511
_BI = 8


def _tc_body(ctab_ref, out_ref):
    ib = pl.program_id(1)
    i0 = ib * _BI
    for u in range(_BI):
        par = (_MAX_LEN - 1 - u) % 2
        k = lax.shift_right_logical(_MAX_LEN - 1 - i0 - u, 1)
        out_ref[0, u] = ctab_ref[pl.ds(par * _PK + k, _MAX_LEN // 2), :]


def _build_tc_kernel(batch):
    return pl.pallas_call(
        _tc_body,
        grid=(batch, _MAX_LEN // _BI),
        in_specs=[pl.BlockSpec((2 * _PK, 2 * _D), lambda b, ib: (0, 0))],
        out_specs=pl.BlockSpec((1, _BI, _MAX_LEN // 2, 2 * _D),
                               lambda b, ib: (b, ib, 0, 0)),
        out_shape=jax.ShapeDtypeStruct(
            (batch, _MAX_LEN, _MAX_LEN // 2, 2 * _D), jnp.float32),
    )


def kernel(x, table):
    batch, seq_len = x.shape
    ftab = jnp.flip(table, axis=0)
    flat = jnp.concatenate(
        [ftab.reshape(-1), jnp.zeros(3 * _D, jnp.float32)])
    even = flat[: _PK * 2 * _D].reshape(_PK, 2 * _D)
    odd = flat[_D: _D + _PK * 2 * _D].reshape(_PK, 2 * _D)
    ctab = jnp.concatenate([even, odd])
    out = _build_tc_kernel(batch)(ctab)
    return out.reshape(batch, seq_len, seq_len, _D)
